# baseline (speedup 1.0000x reference)
import math

import jax
import jax.numpy as jnp
from jax.experimental import pallas as pl
from jax.experimental.pallas import tpu as pltpu

_B, _H, _S, _D = 1, 16, 2048, 64
_N_OUT_TOK = 32
_N_OUT_CH = 8
_QMAX = 7.0
_EPS = 1e-6
_BQ = 512
_SM_SCALE = 1.0 / math.sqrt(_D)


def _topk_mask(score, n, iota, bound):
    def body(_, work):
        m = jnp.max(work)
        idx = jnp.min(jnp.where(work == m, iota, bound))
        return jnp.where(iota == idx, -1.0, work)

    work = jax.lax.fori_loop(0, n, body, score)
    return work < 0.0


def _layer_kernel(q_ref, k_ref, v_ref, o_ref):
    k = k_ref[0]
    v = v_ref[0]

    kscore = jnp.sum(jnp.abs(k), axis=1, keepdims=True)
    iota_s = jax.lax.broadcasted_iota(jnp.int32, (_S, 1), 0)
    k_out = _topk_mask(kscore, _N_OUT_TOK, iota_s, _S)
    k_dense = jnp.where(k_out, 0.0, k)
    k_scale = jnp.max(jnp.abs(k_dense), axis=0, keepdims=True) + _EPS
    k_q = jnp.clip(jnp.round(k_dense / k_scale * _QMAX), -_QMAX, _QMAX)
    k_rec = jnp.where(k_out, k, k_q / _QMAX * k_scale)

    vscore = jnp.sum(jnp.abs(v), axis=0, keepdims=True)
    iota_d = jax.lax.broadcasted_iota(jnp.int32, (1, _D), 1)
    v_out = _topk_mask(vscore, _N_OUT_CH, iota_d, _D)
    v_dense = jnp.where(v_out, 0.0, v)
    v_scale = jnp.max(jnp.abs(v_dense), axis=1, keepdims=True) + _EPS
    v_q = jnp.clip(jnp.round(v_dense / v_scale * _QMAX), -_QMAX, _QMAX)
    v_rec = jnp.where(v_out, v, v_q / _QMAX * v_scale)

    for qb in range(_S // _BQ):
        q = q_ref[0, qb * _BQ:(qb + 1) * _BQ, :]
        s = jax.lax.dot_general(
            q, k_rec, (((1,), (1,)), ((), ())),
            preferred_element_type=jnp.float32) * _SM_SCALE
        m = jnp.max(s, axis=1, keepdims=True)
        p = jnp.exp(s - m)
        p = p / jnp.sum(p, axis=1, keepdims=True)
        o = jax.lax.dot_general(
            p, v_rec, (((1,), (0,)), ((), ())),
            preferred_element_type=jnp.float32)
        o_ref[0, qb * _BQ:(qb + 1) * _BQ, :] = o


def kernel(q_tensor, k_tensor, v_tensor):
    q = q_tensor.reshape(_H, _S, _D)
    k = k_tensor.reshape(_H, _S, _D)
    v = v_tensor.reshape(_H, _S, _D)
    out = pl.pallas_call(
        _layer_kernel,
        grid=(_H,),
        in_specs=[pl.BlockSpec((1, _S, _D), lambda h: (h, 0, 0))] * 3,
        out_specs=pl.BlockSpec((1, _S, _D), lambda h: (h, 0, 0)),
        out_shape=jax.ShapeDtypeStruct((_H, _S, _D), jnp.float32),
        compiler_params=pltpu.CompilerParams(
            dimension_semantics=("parallel",)),
    )(q, k, v)
    return out.reshape(_B, _H, _S, _D)


# speedup vs baseline: 1.7349x; 1.7349x over previous
import math

import jax
import jax.numpy as jnp
from jax.experimental import pallas as pl
from jax.experimental.pallas import tpu as pltpu

_B, _H, _S, _D = 1, 16, 2048, 64
_N_OUT_TOK = 32
_N_OUT_CH = 8
_QMAX = 7.0
_EPS = 1e-6
_BQ = 512
_SM_SCALE = 1.0 / math.sqrt(_D)


def _topk_loop(score, n, iota, bound):
    def body(_, carry):
        work, _, _ = carry
        m = jnp.max(work)
        idx = jnp.min(jnp.where(work == m, iota, bound))
        return jnp.where(iota == idx, -1.0, work), m, idx

    return jax.lax.fori_loop(
        0, n, body, (score, jnp.float32(0.0), jnp.int32(0)))


def _layer_kernel(q_ref, k_ref, v_ref, o_ref):
    k = k_ref[0]
    v = v_ref[0]

    kabs = jnp.abs(k)
    ones_d = jnp.ones((1, _D), dtype=jnp.float32)
    ks_row = jax.lax.dot_general(
        ones_d, kabs, (((1,), (1,)), ((), ())),
        preferred_element_type=jnp.float32)
    iota_row = jax.lax.broadcasted_iota(jnp.int32, (1, _S), 1)
    _, k_thresh, k_last = _topk_loop(ks_row, _N_OUT_TOK, iota_row, _S)
    ks_col = jax.lax.dot_general(
        kabs, ones_d, (((1,), (1,)), ((), ())),
        preferred_element_type=jnp.float32)
    iota_col = jax.lax.broadcasted_iota(jnp.int32, (_S, 1), 0)
    k_out = (ks_col > k_thresh) | ((ks_col == k_thresh)
                                   & (iota_col <= k_last))
    k_dense = jnp.where(k_out, 0.0, k)
    k_scale = jnp.max(jnp.abs(k_dense), axis=0, keepdims=True) + _EPS
    k_q = jnp.clip(jnp.round(k_dense * (_QMAX / k_scale)), -_QMAX, _QMAX)
    k_rec = jnp.where(k_out, k, k_q * (k_scale * (1.0 / _QMAX)))

    vscore = jnp.sum(jnp.abs(v), axis=0, keepdims=True)
    iota_d = jax.lax.broadcasted_iota(jnp.int32, (1, _D), 1)
    vw, _, _ = _topk_loop(vscore, _N_OUT_CH, iota_d, _D)
    v_out = vw < 0.0
    v_dense = jnp.where(v_out, 0.0, v)
    v_scale = jnp.max(jnp.abs(v_dense), axis=1, keepdims=True) + _EPS
    v_q = jnp.clip(jnp.round(v_dense / v_scale * _QMAX), -_QMAX, _QMAX)
    v_rec = jnp.where(v_out, v, v_q / _QMAX * v_scale)
    v_aug = jnp.concatenate(
        [v_rec, jnp.ones((_S, 1), dtype=jnp.float32)], axis=1)

    for qb in range(_S // _BQ):
        q = q_ref[0, qb * _BQ:(qb + 1) * _BQ, :] * _SM_SCALE
        s = jax.lax.dot_general(
            q, k_rec, (((1,), (1,)), ((), ())),
            preferred_element_type=jnp.float32)
        p = jnp.exp(s)
        o_aug = jax.lax.dot_general(
            p, v_aug, (((1,), (0,)), ((), ())),
            preferred_element_type=jnp.float32)
        o = o_aug[:, :_D] * (1.0 / o_aug[:, _D:_D + 1])
        o_ref[0, qb * _BQ:(qb + 1) * _BQ, :] = o


def kernel(q_tensor, k_tensor, v_tensor):
    q = q_tensor.reshape(_H, _S, _D)
    k = k_tensor.reshape(_H, _S, _D)
    v = v_tensor.reshape(_H, _S, _D)
    out = pl.pallas_call(
        _layer_kernel,
        grid=(_H,),
        in_specs=[pl.BlockSpec((1, _S, _D), lambda h: (h, 0, 0))] * 3,
        out_specs=pl.BlockSpec((1, _S, _D), lambda h: (h, 0, 0)),
        out_shape=jax.ShapeDtypeStruct((_H, _S, _D), jnp.float32),
        compiler_params=pltpu.CompilerParams(
            dimension_semantics=("parallel",)),
    )(q, k, v)
    return out.reshape(_B, _H, _S, _D)


# speedup vs baseline: 1.7404x; 1.0032x over previous
import math

import jax
import jax.numpy as jnp
from jax.experimental import pallas as pl
from jax.experimental.pallas import tpu as pltpu

_B, _H, _S, _D = 1, 16, 2048, 64
_N_OUT_TOK = 32
_N_OUT_CH = 8
_QMAX = 7.0
_EPS = 1e-6
_BQ = 512
_SM_SCALE = 1.0 / math.sqrt(_D)


def _topk_loop(score, n, iota, bound, mark_ref=None):
    def body(_, work):
        m = jnp.max(work)
        idx = jnp.min(jnp.where(work == m, iota, bound))
        if mark_ref is not None:
            mark_ref[pl.ds(idx, 1), :] = jnp.ones((1, 1), jnp.float32)
        return jnp.where(iota == idx, -1.0, work)

    return jax.lax.fori_loop(0, n, body, score)


def _layer_kernel(q_ref, k_ref, v_ref, o_ref, kout_ref):
    k = k_ref[0]
    v = v_ref[0]

    kabs = jnp.abs(k)
    ones_d = jnp.ones((1, _D), dtype=jnp.float32)
    ks_row = jax.lax.dot_general(
        ones_d, kabs, (((1,), (1,)), ((), ())),
        preferred_element_type=jnp.float32)
    iota_row = jax.lax.broadcasted_iota(jnp.int32, (1, _S), 1)
    kout_ref[...] = jnp.zeros((_S, 1), jnp.float32)
    _topk_loop(ks_row, _N_OUT_TOK, iota_row, _S, mark_ref=kout_ref)
    k_out = kout_ref[...] > 0.0
    k_dense = jnp.where(k_out, 0.0, k)
    k_scale = jnp.max(jnp.abs(k_dense), axis=0, keepdims=True) + _EPS
    k_q = jnp.clip(jnp.round(k_dense * (_QMAX / k_scale)), -_QMAX, _QMAX)
    k_rec = jnp.where(k_out, k, k_q * (k_scale * (1.0 / _QMAX)))

    vscore = jnp.sum(jnp.abs(v), axis=0, keepdims=True)
    iota_d = jax.lax.broadcasted_iota(jnp.int32, (1, _D), 1)
    vw = _topk_loop(vscore, _N_OUT_CH, iota_d, _D)
    v_out = vw < 0.0
    v_dense = jnp.where(v_out, 0.0, v)
    v_scale = jnp.max(jnp.abs(v_dense), axis=1, keepdims=True) + _EPS
    v_q = jnp.clip(jnp.round(v_dense / v_scale * _QMAX), -_QMAX, _QMAX)
    v_rec = jnp.where(v_out, v, v_q / _QMAX * v_scale)
    v_aug = jnp.concatenate(
        [v_rec, jnp.ones((_S, 1), dtype=jnp.float32)], axis=1)

    for qb in range(_S // _BQ):
        q = q_ref[0, qb * _BQ:(qb + 1) * _BQ, :] * _SM_SCALE
        s = jax.lax.dot_general(
            q, k_rec, (((1,), (1,)), ((), ())),
            preferred_element_type=jnp.float32)
        p = jnp.exp(s)
        o_aug = jax.lax.dot_general(
            p, v_aug, (((1,), (0,)), ((), ())),
            preferred_element_type=jnp.float32)
        o = o_aug[:, :_D] * (1.0 / o_aug[:, _D:_D + 1])
        o_ref[0, qb * _BQ:(qb + 1) * _BQ, :] = o


def kernel(q_tensor, k_tensor, v_tensor):
    q = q_tensor.reshape(_H, _S, _D)
    k = k_tensor.reshape(_H, _S, _D)
    v = v_tensor.reshape(_H, _S, _D)
    out = pl.pallas_call(
        _layer_kernel,
        grid=(_H,),
        in_specs=[pl.BlockSpec((1, _S, _D), lambda h: (h, 0, 0))] * 3,
        out_specs=pl.BlockSpec((1, _S, _D), lambda h: (h, 0, 0)),
        out_shape=jax.ShapeDtypeStruct((_H, _S, _D), jnp.float32),
        scratch_shapes=[pltpu.VMEM((_S, 1), jnp.float32)],
        compiler_params=pltpu.CompilerParams(
            dimension_semantics=("parallel",)),
    )(q, k, v)
    return out.reshape(_B, _H, _S, _D)


# speedup vs baseline: 1.7775x; 1.0213x over previous
import math

import jax
import jax.numpy as jnp
from jax.experimental import pallas as pl
from jax.experimental.pallas import tpu as pltpu

_B, _H, _S, _D = 1, 16, 2048, 64
_N_OUT_TOK = 32
_N_OUT_CH = 8
_QMAX = 7.0
_EPS = 1e-6
_BQ = 512
_SM_SCALE = 1.0 / math.sqrt(_D)


def _topk_loop(score, n, iota, bound, mark_ref=None):
    def body(_, work):
        m = jnp.max(work)
        idx = jnp.min(jnp.where(work == m, iota, bound))
        if mark_ref is not None:
            mark_ref[pl.ds(idx, 1), :] = jnp.ones((1, 1), jnp.float32)
        return jnp.where(iota == idx, -1.0, work)

    return jax.lax.fori_loop(0, n, body, score)


def _layer_kernel(q_ref, k_ref, v_ref, o_ref, kout_ref):
    k = k_ref[0, 0]
    v = v_ref[0, 0]

    kabs = jnp.abs(k)
    ones_d = jnp.ones((1, _D), dtype=jnp.float32)
    ks_row = jax.lax.dot_general(
        ones_d, kabs, (((1,), (1,)), ((), ())),
        preferred_element_type=jnp.float32)
    iota_row = jax.lax.broadcasted_iota(jnp.int32, (1, _S), 1)
    kout_ref[...] = jnp.zeros((_S, 1), jnp.float32)
    _topk_loop(ks_row, _N_OUT_TOK, iota_row, _S, mark_ref=kout_ref)
    k_out = kout_ref[...] > 0.0
    k_dense = jnp.where(k_out, 0.0, k)
    k_scale = jnp.max(jnp.abs(k_dense), axis=0, keepdims=True) + _EPS
    k_q = jnp.clip(jnp.round(k_dense * (_QMAX / k_scale)), -_QMAX, _QMAX)
    k_rec = jnp.where(k_out, k, k_q * (k_scale * (1.0 / _QMAX)))

    vscore = jnp.sum(jnp.abs(v), axis=0, keepdims=True)
    iota_d = jax.lax.broadcasted_iota(jnp.int32, (1, _D), 1)
    vw = _topk_loop(vscore, _N_OUT_CH, iota_d, _D)
    v_out = vw < 0.0
    v_dense = jnp.where(v_out, 0.0, v)
    v_scale = jnp.max(jnp.abs(v_dense), axis=1, keepdims=True) + _EPS
    v_q = jnp.clip(jnp.round(v_dense / v_scale * _QMAX), -_QMAX, _QMAX)
    v_rec = jnp.where(v_out, v, v_q / _QMAX * v_scale)
    v_aug = jnp.concatenate(
        [v_rec, jnp.ones((_S, 1), dtype=jnp.float32)], axis=1)

    for qb in range(_S // _BQ):
        q = q_ref[0, 0, qb * _BQ:(qb + 1) * _BQ, :] * _SM_SCALE
        s = jax.lax.dot_general(
            q, k_rec, (((1,), (1,)), ((), ())),
            preferred_element_type=jnp.float32)
        p = jnp.exp(s)
        o_aug = jax.lax.dot_general(
            p, v_aug, (((1,), (0,)), ((), ())),
            preferred_element_type=jnp.float32)
        o = o_aug[:, :_D] * (1.0 / o_aug[:, _D:_D + 1])
        o_ref[0, 0, qb * _BQ:(qb + 1) * _BQ, :] = o


def kernel(q_tensor, k_tensor, v_tensor):
    return pl.pallas_call(
        _layer_kernel,
        grid=(_H,),
        in_specs=[pl.BlockSpec((1, 1, _S, _D), lambda h: (0, h, 0, 0))] * 3,
        out_specs=pl.BlockSpec((1, 1, _S, _D), lambda h: (0, h, 0, 0)),
        out_shape=jax.ShapeDtypeStruct((_B, _H, _S, _D), jnp.float32),
        scratch_shapes=[pltpu.VMEM((_S, 1), jnp.float32)],
        compiler_params=pltpu.CompilerParams(
            dimension_semantics=("parallel",)),
    )(q_tensor, k_tensor, v_tensor)


# speedup vs baseline: 3.8651x; 2.1744x over previous
import math

import jax
import jax.numpy as jnp
from jax.experimental import pallas as pl
from jax.experimental.pallas import tpu as pltpu

_B, _H, _S, _D = 1, 16, 2048, 64
_N_OUT_TOK = 32
_N_OUT_CH = 8
_QMAX = 7.0
_EPS = 1e-6
_BQ = 512
_SM_SCALE = 1.0 / math.sqrt(_D)


def _topk_rows(score, n, width):
    iota = jax.lax.broadcasted_iota(jnp.int32, score.shape, 1)

    def body(_, work):
        m = jnp.max(work, axis=1, keepdims=True)
        idx = jnp.min(jnp.where(work == m, iota, width),
                      axis=1, keepdims=True)
        return jnp.where(iota == idx, -1.0, work)

    return jax.lax.fori_loop(0, n, body, score)


def _mask_kernel(k_ref, v_ref, km_ref, vm_ref):
    k = k_ref[0]
    v = v_ref[0]
    ones_d = jnp.ones((1, _D), dtype=jnp.float32)
    ones_s = jnp.ones((1, _S), dtype=jnp.float32)
    ks = jnp.concatenate(
        [jax.lax.dot_general(ones_d, jnp.abs(k[h]), (((1,), (1,)), ((), ())),
                             preferred_element_type=jnp.float32)
         for h in range(_H)], axis=0)
    vs = jnp.concatenate(
        [jax.lax.dot_general(ones_s, jnp.abs(v[h]), (((1,), (0,)), ((), ())),
                             preferred_element_type=jnp.float32)
         for h in range(_H)], axis=0)
    kw = _topk_rows(ks, _N_OUT_TOK, _S)
    vw = _topk_rows(vs, _N_OUT_CH, _D)
    eye_h = jnp.float32(
        jax.lax.broadcasted_iota(jnp.int32, (_H, _H), 0)
        == jax.lax.broadcasted_iota(jnp.int32, (_H, _H), 1))
    km_ref[...] = jax.lax.dot_general(
        kw, eye_h, (((0,), (0,)), ((), ())),
        preferred_element_type=jnp.float32)
    vm_ref[...] = vw


def _layer_kernel(q_ref, k_ref, v_ref, km_ref, vm_ref, o_ref):
    h = pl.program_id(0)
    k = k_ref[0, 0]
    v = v_ref[0, 0]
    onehot_h = jnp.float32(
        jax.lax.broadcasted_iota(jnp.int32, (_H, 1), 0) == h)
    k_out = jax.lax.dot_general(
        km_ref[...], onehot_h, (((1,), (0,)), ((), ())),
        preferred_element_type=jnp.float32) < 0.0
    v_out = vm_ref[pl.ds(h, 1), :] < 0.0

    k_dense = jnp.where(k_out, 0.0, k)
    k_scale = jnp.max(jnp.abs(k_dense), axis=0, keepdims=True) + _EPS
    k_q = jnp.clip(jnp.round(k_dense * (_QMAX / k_scale)), -_QMAX, _QMAX)
    k_rec = jnp.where(k_out, k, k_q * (k_scale * (1.0 / _QMAX)))

    v_dense = jnp.where(v_out, 0.0, v)
    v_scale = jnp.max(jnp.abs(v_dense), axis=1, keepdims=True) + _EPS
    v_q = jnp.clip(jnp.round(v_dense / v_scale * _QMAX), -_QMAX, _QMAX)
    v_rec = jnp.where(v_out, v, v_q / _QMAX * v_scale)
    v_aug = jnp.concatenate(
        [v_rec, jnp.ones((_S, 1), dtype=jnp.float32)], axis=1)

    for qb in range(_S // _BQ):
        q = q_ref[0, 0, qb * _BQ:(qb + 1) * _BQ, :] * _SM_SCALE
        s = jax.lax.dot_general(
            q, k_rec, (((1,), (1,)), ((), ())),
            preferred_element_type=jnp.float32)
        p = jnp.exp(s)
        o_aug = jax.lax.dot_general(
            p, v_aug, (((1,), (0,)), ((), ())),
            preferred_element_type=jnp.float32)
        o = o_aug[:, :_D] * (1.0 / o_aug[:, _D:_D + 1])
        o_ref[0, 0, qb * _BQ:(qb + 1) * _BQ, :] = o


def kernel(q_tensor, k_tensor, v_tensor):
    km, vm = pl.pallas_call(
        _mask_kernel,
        grid=(1,),
        in_specs=[
            pl.BlockSpec((1, _H, _S, _D), lambda i: (0, 0, 0, 0)),
            pl.BlockSpec((1, _H, _S, _D), lambda i: (0, 0, 0, 0)),
        ],
        out_specs=[
            pl.BlockSpec((_S, _H), lambda i: (0, 0)),
            pl.BlockSpec((_H, _D), lambda i: (0, 0)),
        ],
        out_shape=[
            jax.ShapeDtypeStruct((_S, _H), jnp.float32),
            jax.ShapeDtypeStruct((_H, _D), jnp.float32),
        ],
    )(k_tensor, v_tensor)

    return pl.pallas_call(
        _layer_kernel,
        grid=(_H,),
        in_specs=[
            pl.BlockSpec((1, 1, _S, _D), lambda h: (0, h, 0, 0)),
            pl.BlockSpec((1, 1, _S, _D), lambda h: (0, h, 0, 0)),
            pl.BlockSpec((1, 1, _S, _D), lambda h: (0, h, 0, 0)),
            pl.BlockSpec((_S, _H), lambda h: (0, 0)),
            pl.BlockSpec((_H, _D), lambda h: (0, 0)),
        ],
        out_specs=pl.BlockSpec((1, 1, _S, _D), lambda h: (0, h, 0, 0)),
        out_shape=jax.ShapeDtypeStruct((_B, _H, _S, _D), jnp.float32),
        compiler_params=pltpu.CompilerParams(
            dimension_semantics=("parallel",)),
    )(q_tensor, k_tensor, v_tensor, km, vm)


# speedup vs baseline: 6.2652x; 1.6210x over previous
import math

import jax
import jax.numpy as jnp
from jax.experimental import pallas as pl
from jax.experimental.pallas import tpu as pltpu

_B, _H, _S, _D = 1, 16, 2048, 64
_N_OUT_TOK = 32
_N_OUT_CH = 8
_QMAX = 7.0
_EPS = 1e-6
_BQ = 512
_SM_SCALE = 1.0 / math.sqrt(_D)


def _topk_rows(score, n, width):
    iota = jax.lax.broadcasted_iota(jnp.int32, score.shape, 1)

    def body(_, work):
        m = jnp.max(work, axis=1, keepdims=True)
        idx = jnp.min(jnp.where(work == m, iota, width),
                      axis=1, keepdims=True)
        return jnp.where(iota == idx, -1.0, work)

    return jax.lax.fori_loop(0, n, body, score)


def _mask_kernel(k_ref, v_ref, km_ref, vm_ref):
    k = k_ref[0]
    v = v_ref[0]
    ones_d = jnp.ones((1, _D), dtype=jnp.float32)
    ones_s = jnp.ones((1, _S), dtype=jnp.float32)
    ks = jnp.concatenate(
        [jax.lax.dot_general(ones_d, jnp.abs(k[h]), (((1,), (0,)), ((), ())),
                             preferred_element_type=jnp.float32)
         for h in range(_H)], axis=0)
    vs = jnp.concatenate(
        [jax.lax.dot_general(ones_s, jnp.abs(v[h]), (((1,), (1,)), ((), ())),
                             preferred_element_type=jnp.float32)
         for h in range(_H)], axis=0)
    kw = _topk_rows(ks, _N_OUT_TOK, _S)
    vw = _topk_rows(vs, _N_OUT_CH, _D)
    km_ref[...] = kw
    eye_h = jnp.float32(
        jax.lax.broadcasted_iota(jnp.int32, (_H, _H), 0)
        == jax.lax.broadcasted_iota(jnp.int32, (_H, _H), 1))
    vm_ref[...] = jax.lax.dot_general(
        vw, eye_h, (((0,), (0,)), ((), ())),
        preferred_element_type=jnp.float32)


def _layer_kernel(q_ref, k_ref, v_ref, km_ref, vm_ref, o_ref):
    h = pl.program_id(0)
    k = k_ref[0, 0]
    v = v_ref[0, 0]
    k_out = km_ref[pl.ds(h, 1), :] < 0.0
    onehot_h = jnp.float32(
        jax.lax.broadcasted_iota(jnp.int32, (_H, 1), 0) == h)
    v_out = jax.lax.dot_general(
        vm_ref[...], onehot_h, (((1,), (0,)), ((), ())),
        preferred_element_type=jnp.float32) < 0.0

    k_dense = jnp.where(k_out, 0.0, k)
    k_scale = jnp.max(jnp.abs(k_dense), axis=1, keepdims=True) + _EPS
    k_q = jnp.clip(jnp.round(k_dense * (_QMAX / k_scale)), -_QMAX, _QMAX)
    k_rec = jnp.where(k_out, k, k_q * (k_scale * (1.0 / _QMAX)))

    v_dense = jnp.where(v_out, 0.0, v)
    v_scale = jnp.max(jnp.abs(v_dense), axis=0, keepdims=True) + _EPS
    v_q = jnp.clip(jnp.round(v_dense / v_scale * _QMAX), -_QMAX, _QMAX)
    v_rec = jnp.where(v_out, v, v_q / _QMAX * v_scale)
    v_aug = jnp.concatenate(
        [v_rec, jnp.ones((1, _S), dtype=jnp.float32)], axis=0)

    for qb in range(_S // _BQ):
        q = q_ref[0, 0, :, qb * _BQ:(qb + 1) * _BQ] * _SM_SCALE
        s = jax.lax.dot_general(
            q, k_rec, (((0,), (0,)), ((), ())),
            preferred_element_type=jnp.float32)
        p = jnp.exp(s)
        o_aug = jax.lax.dot_general(
            v_aug, p, (((1,), (1,)), ((), ())),
            preferred_element_type=jnp.float32)
        o = o_aug[:_D, :] * (1.0 / o_aug[_D:_D + 1, :])
        o_ref[0, 0, :, qb * _BQ:(qb + 1) * _BQ] = o


def kernel(q_tensor, k_tensor, v_tensor):
    qt = jnp.transpose(q_tensor, (0, 1, 3, 2))
    kt = jnp.transpose(k_tensor, (0, 1, 3, 2))
    vt = jnp.transpose(v_tensor, (0, 1, 3, 2))

    km, vm = pl.pallas_call(
        _mask_kernel,
        grid=(1,),
        in_specs=[
            pl.BlockSpec((1, _H, _D, _S), lambda i: (0, 0, 0, 0)),
            pl.BlockSpec((1, _H, _D, _S), lambda i: (0, 0, 0, 0)),
        ],
        out_specs=[
            pl.BlockSpec((_H, _S), lambda i: (0, 0)),
            pl.BlockSpec((_D, _H), lambda i: (0, 0)),
        ],
        out_shape=[
            jax.ShapeDtypeStruct((_H, _S), jnp.float32),
            jax.ShapeDtypeStruct((_D, _H), jnp.float32),
        ],
    )(kt, vt)

    out = pl.pallas_call(
        _layer_kernel,
        grid=(_H,),
        in_specs=[
            pl.BlockSpec((1, 1, _D, _S), lambda h: (0, h, 0, 0)),
            pl.BlockSpec((1, 1, _D, _S), lambda h: (0, h, 0, 0)),
            pl.BlockSpec((1, 1, _D, _S), lambda h: (0, h, 0, 0)),
            pl.BlockSpec((_H, _S), lambda h: (0, 0)),
            pl.BlockSpec((_D, _H), lambda h: (0, 0)),
        ],
        out_specs=pl.BlockSpec((1, 1, _D, _S), lambda h: (0, h, 0, 0)),
        out_shape=jax.ShapeDtypeStruct((_B, _H, _D, _S), jnp.float32),
        compiler_params=pltpu.CompilerParams(
            dimension_semantics=("parallel",)),
    )(qt, kt, vt, km, vm)
    return jnp.transpose(out, (0, 1, 3, 2))
